# baseline (speedup 1.0000x reference)
import functools

import jax
import jax.numpy as jnp
from jax import lax
from jax.experimental import pallas as pl
from jax.experimental.pallas import tpu as pltpu
from jax.experimental.pallas import tpu_sc as plsc

NUM_EMB = 1000000
FEAT = 32
N_B = 16384
N_J = 26
B_TOTAL = N_B * N_J
IDX_W = 128
N_ROWS = B_TOTAL // IDX_W
NC, NS = 2, 16
NW = NC * NS
L = 16

C_BLK = 512
N_CBLK = NUM_EMB // C_BLK + 1
RB = (N_CBLK + NW - 1) // NW

B_BLK = 512
GPT = B_BLK // IDX_W
N_BLK = N_B // B_BLK
N_TASKS = N_J * N_BLK
TPW = N_TASKS // NW
K = TPW * GPT

_SC_PARAMS = pltpu.CompilerParams(
    use_tc_tiling_on_sc=False, needs_layout_passes=False
)


@functools.partial(
    pl.kernel,
    out_type=jax.ShapeDtypeStruct((NUM_EMB, FEAT), jnp.float32),
    mesh=plsc.VectorSubcoreMesh(core_axis_name="c", subcore_axis_name="s"),
    scratch_types=[
        pltpu.VMEM((2, FEAT, C_BLK), jnp.float32),
        pltpu.VMEM((2, C_BLK, FEAT), jnp.float32),
        pltpu.SemaphoreType.DMA,
        pltpu.SemaphoreType.DMA,
        pltpu.SemaphoreType.DMA,
    ],
    compiler_params=_SC_PARAMS,
)
def _repack_sc(tab_t_hbm, out_hbm, cin_v, cout_v, sem_i, sem_o0, sem_o1):
    wid = lax.axis_index("s") * NC + lax.axis_index("c")
    sems = (sem_o0, sem_o1)

    def c0_of(k):
        m = wid + NW * k
        c0 = jnp.where(
            m == N_CBLK - 1,
            NUM_EMB - C_BLK,
            jnp.where(m >= N_CBLK, (m - NW) * C_BLK, m * C_BLK),
        )
        return pl.multiple_of(c0, 8)

    def fire_in(k, buf):
        pltpu.async_copy(
            tab_t_hbm.at[:, pl.ds(c0_of(k), C_BLK)], cin_v.at[buf], sem_i
        )

    def drain_in():
        pltpu.make_async_copy(
            tab_t_hbm.at[:, pl.ds(0, C_BLK)], cin_v.at[0], sem_i
        ).wait()

    f_lo = lax.iota(jnp.int32, L)
    f_hi = f_lo + L

    def transpose(buf):
        cin2 = cin_v.at[buf]

        @plsc.parallel_loop(0, C_BLK, unroll=4)
        def _(r):
            rs = jnp.full((L,), r, jnp.int32)
            x0 = plsc.load_gather(cin2, [f_lo, rs])
            x1 = plsc.load_gather(cin2, [f_hi, rs])
            cout_v[buf, r, pl.ds(0, L)] = x0
            cout_v[buf, r, pl.ds(L, L)] = x1

    def fire_out(k, buf):
        pltpu.async_copy(
            cout_v.at[buf], out_hbm.at[pl.ds(c0_of(k), C_BLK)], sems[buf]
        )

    def drain_out(buf):
        pltpu.make_async_copy(
            cout_v.at[0], out_hbm.at[pl.ds(0, C_BLK)], sems[buf]
        ).wait()

    fire_in(0, 0)
    drain_in()
    fire_in(1, 1)
    transpose(0)
    fire_out(0, 0)

    drain_in()
    fire_in(2, 0)
    transpose(1)
    fire_out(1, 1)

    def pair(p, carry):
        k = 2 * p + 2
        drain_in()
        fire_in(k + 1, 1)
        drain_out(0)
        transpose(0)
        fire_out(k, 0)

        drain_in()
        fire_in(k + 2, 0)
        drain_out(1)
        transpose(1)
        fire_out(k + 1, 1)
        return carry

    lax.fori_loop(0, (RB - 4) // 2, pair, 0)

    drain_in()
    fire_in(RB - 1, 1)
    drain_out(0)
    transpose(0)
    fire_out(RB - 2, 0)

    drain_in()
    drain_out(1)
    transpose(1)
    fire_out(RB - 1, 1)

    drain_out(0)
    drain_out(1)


@functools.partial(
    pl.kernel,
    out_type=jax.ShapeDtypeStruct((N_J, FEAT, N_B), jnp.float32),
    mesh=plsc.VectorSubcoreMesh(core_axis_name="c", subcore_axis_name="s"),
    scratch_types=[
        pltpu.VMEM((K, IDX_W), jnp.int32),
        pltpu.VMEM((2, B_BLK, FEAT), jnp.float32),
        pltpu.VMEM((2, FEAT, B_BLK), jnp.float32),
        pltpu.SemaphoreType.DMA,
        pltpu.SemaphoreType.DMA,
        pltpu.SemaphoreType.DMA,
    ],
    compiler_params=_SC_PARAMS,
)
def _embed_sc(
    idx_hbm, table_hbm, out_hbm, idx_v, rows_v, tr_v, sem_g, sem_o0, sem_o1
):
    wid = lax.axis_index("s") * NC + lax.axis_index("c")
    pltpu.sync_copy(idx_hbm.at[pl.ds(wid * K, K)], idx_v)
    t0 = wid * TPW
    sems = (sem_o0, sem_o1)

    def fire(i, buf):
        for r in range(GPT):
            pltpu.async_copy(
                table_hbm.at[idx_v.at[i * GPT + r]],
                rows_v.at[buf, pl.ds(r * IDX_W, IDX_W)],
                sem_g,
            )

    def drain_gathers():
        pltpu.make_async_copy(
            table_hbm.at[pl.ds(0, B_BLK)], rows_v.at[0], sem_g
        ).wait()

    f_lo = lax.iota(jnp.int32, L)
    f_hi = f_lo + L

    def transpose(buf):
        tr2 = tr_v.at[buf]

        @plsc.parallel_loop(0, B_BLK, unroll=4)
        def _(b):
            b_sp = jnp.full((L,), b, jnp.int32)
            x0 = rows_v[buf, b, pl.ds(0, L)]
            x1 = rows_v[buf, b, pl.ds(L, L)]
            plsc.store_scatter(tr2, [f_lo, b_sp], x0)
            plsc.store_scatter(tr2, [f_hi, b_sp], x1)

    def start_out(i, buf):
        t = t0 + i
        j = t // N_BLK
        blk = t % N_BLK
        pltpu.async_copy(
            tr_v.at[buf],
            out_hbm.at[j, :, pl.ds(blk * B_BLK, B_BLK)],
            sems[buf],
        )

    def drain_out(buf):
        pltpu.make_async_copy(
            tr_v.at[0], out_hbm.at[0, :, pl.ds(0, B_BLK)], sems[buf]
        ).wait()

    fire(0, 0)
    drain_gathers()
    fire(1, 1)
    transpose(0)
    start_out(0, 0)

    drain_gathers()
    fire(2, 0)
    transpose(1)
    start_out(1, 1)

    def pair(p, carry):
        i = 2 * p + 2
        drain_gathers()
        fire(i + 1, 1)
        drain_out(0)
        transpose(0)
        start_out(i, 0)

        drain_gathers()
        fire(i + 2, 0)
        drain_out(1)
        transpose(1)
        start_out(i + 1, 1)
        return carry

    lax.fori_loop(0, (TPW - 4) // 2, pair, 0)

    drain_gathers()
    fire(TPW - 1, 1)
    drain_out(0)
    transpose(0)
    start_out(TPW - 2, 0)

    drain_gathers()
    drain_out(1)
    transpose(1)
    start_out(TPW - 1, 1)

    drain_out(0)
    drain_out(1)


def kernel(inputs, embedding):
    idx2d = inputs.T.reshape(N_ROWS, IDX_W)
    table_rm = _repack_sc(embedding.T)
    out = _embed_sc(idx2d, table_rm)
    return jnp.transpose(out, (2, 0, 1))


# speedup vs baseline: 4.3470x; 4.3470x over previous
import functools

import jax
import jax.numpy as jnp
from jax import lax
from jax.experimental import pallas as pl
from jax.experimental.pallas import tpu as pltpu
from jax.experimental.pallas import tpu_sc as plsc

NUM_EMB = 1000000
FEAT = 32
PACK = 4
N_B = 16384
N_J = 26
B_TOTAL = N_B * N_J
IDX_W = 128
N_ROWS = B_TOTAL // IDX_W
NC, NS = 2, 16
NW = NC * NS
L = 16

C_BLK = 512
N_CBLK = NUM_EMB // C_BLK
RB = (N_CBLK + NW - 1) // NW
TAIL_C0 = N_CBLK * C_BLK
TAIL_C = NUM_EMB - TAIL_C0

B_BLK = 512
GPT = B_BLK // IDX_W
N_BLK = N_B // B_BLK
N_TASKS = N_J * N_BLK
TPW = N_TASKS // NW
K = TPW * GPT

_SC_PARAMS = pltpu.CompilerParams(
    use_tc_tiling_on_sc=False, needs_layout_passes=False
)


@functools.partial(
    pl.kernel,
    out_type=jax.ShapeDtypeStruct((NUM_EMB // PACK, PACK * FEAT), jnp.float32),
    mesh=plsc.VectorSubcoreMesh(core_axis_name="c", subcore_axis_name="s"),
    scratch_types=[
        pltpu.VMEM((2, FEAT, C_BLK), jnp.float32),
        pltpu.VMEM((2, C_BLK // PACK, PACK * FEAT), jnp.float32),
        pltpu.SemaphoreType.DMA,
        pltpu.SemaphoreType.DMA,
        pltpu.SemaphoreType.DMA,
    ],
    compiler_params=pltpu.CompilerParams(
        use_tc_tiling_on_sc=True, needs_layout_passes=False
    ),
)
def _repack_sc(tab_t_hbm, tail_hbm, out_hbm, cin_v, cout_v, sem_i, sem_o0, sem_o1):
    wid = lax.axis_index("s") * NC + lax.axis_index("c")
    sems = (sem_o0, sem_o1)

    def c0_of(k):
        m = wid + NW * k
        m = jnp.where(m >= N_CBLK, m - NW, m)
        return m * C_BLK

    def fire_in(k, buf):
        pltpu.async_copy(
            tab_t_hbm.at[:, pl.ds(pl.multiple_of(c0_of(k), 128), C_BLK)],
            cin_v.at[buf],
            sem_i,
        )

    def drain_in():
        pltpu.make_async_copy(
            tab_t_hbm.at[:, pl.ds(0, C_BLK)], cin_v.at[0], sem_i
        ).wait()

    f_lo = lax.iota(jnp.int32, L)
    f_hi = f_lo + L

    def transpose(buf):
        cin2 = cin_v.at[buf]

        @plsc.parallel_loop(0, C_BLK, unroll=4)
        def _(r):
            rs = jnp.full((L,), r, jnp.int32)
            x0 = plsc.load_gather(cin2, [f_lo, rs])
            x1 = plsc.load_gather(cin2, [f_hi, rs])
            pr = r // PACK
            base = (r % PACK) * FEAT
            cout_v[buf, pr, pl.ds(base, L)] = x0
            cout_v[buf, pr, pl.ds(base + L, L)] = x1

    def fire_out(k, buf):
        pltpu.async_copy(
            cout_v.at[buf],
            out_hbm.at[pl.ds(pl.multiple_of(c0_of(k) // PACK, 8), C_BLK // PACK)],
            sems[buf],
        )

    def drain_out(buf):
        pltpu.make_async_copy(
            cout_v.at[0], out_hbm.at[pl.ds(0, C_BLK // PACK)], sems[buf]
        ).wait()

    fire_in(0, 0)
    drain_in()
    fire_in(1, 1)
    transpose(0)
    fire_out(0, 0)

    drain_in()
    fire_in(2, 0)
    transpose(1)
    fire_out(1, 1)

    def pair(p, carry):
        k = 2 * p + 2
        drain_in()
        fire_in(k + 1, 1)
        drain_out(0)
        transpose(0)
        fire_out(k, 0)

        drain_in()
        fire_in(k + 2, 0)
        drain_out(1)
        transpose(1)
        fire_out(k + 1, 1)
        return carry

    lax.fori_loop(0, (RB - 4) // 2, pair, 0)

    drain_in()
    fire_in(RB - 1, 1)
    drain_out(0)
    transpose(0)
    fire_out(RB - 2, 0)

    drain_in()
    drain_out(1)
    transpose(1)
    fire_out(RB - 1, 1)

    drain_out(0)
    drain_out(1)

    @pl.when(wid == 0)
    def _tail():
        pltpu.sync_copy(
            tail_hbm,
            out_hbm.at[pl.ds(TAIL_C0 // PACK, TAIL_C // PACK)],
        )


@functools.partial(
    pl.kernel,
    out_type=jax.ShapeDtypeStruct((N_J, FEAT, N_B), jnp.float32),
    mesh=plsc.VectorSubcoreMesh(core_axis_name="c", subcore_axis_name="s"),
    scratch_types=[
        pltpu.VMEM((K, IDX_W), jnp.int32),
        pltpu.VMEM((2, B_BLK, FEAT), jnp.float32),
        pltpu.VMEM((2, FEAT, B_BLK), jnp.float32),
        pltpu.SemaphoreType.DMA,
        pltpu.SemaphoreType.DMA,
        pltpu.SemaphoreType.DMA,
    ],
    compiler_params=_SC_PARAMS,
)
def _embed_sc(
    idx_hbm, table_hbm, out_hbm, idx_v, rows_v, tr_v, sem_g, sem_o0, sem_o1
):
    wid = lax.axis_index("s") * NC + lax.axis_index("c")
    pltpu.sync_copy(idx_hbm.at[pl.ds(wid * K, K)], idx_v)
    t0 = wid * TPW
    sems = (sem_o0, sem_o1)

    def fire(i, buf):
        for r in range(GPT):
            pltpu.async_copy(
                table_hbm.at[idx_v.at[i * GPT + r]],
                rows_v.at[buf, pl.ds(r * IDX_W, IDX_W)],
                sem_g,
            )

    def drain_gathers():
        pltpu.make_async_copy(
            table_hbm.at[pl.ds(0, B_BLK)], rows_v.at[0], sem_g
        ).wait()

    f_lo = lax.iota(jnp.int32, L)
    f_hi = f_lo + L

    def transpose(buf):
        tr2 = tr_v.at[buf]

        @plsc.parallel_loop(0, B_BLK, unroll=4)
        def _(b):
            b_sp = jnp.full((L,), b, jnp.int32)
            x0 = rows_v[buf, b, pl.ds(0, L)]
            x1 = rows_v[buf, b, pl.ds(L, L)]
            plsc.store_scatter(tr2, [f_lo, b_sp], x0)
            plsc.store_scatter(tr2, [f_hi, b_sp], x1)

    def start_out(i, buf):
        t = t0 + i
        j = t // N_BLK
        blk = t % N_BLK
        pltpu.async_copy(
            tr_v.at[buf],
            out_hbm.at[j, :, pl.ds(blk * B_BLK, B_BLK)],
            sems[buf],
        )

    def drain_out(buf):
        pltpu.make_async_copy(
            tr_v.at[0], out_hbm.at[0, :, pl.ds(0, B_BLK)], sems[buf]
        ).wait()

    fire(0, 0)
    drain_gathers()
    fire(1, 1)
    transpose(0)
    start_out(0, 0)

    drain_gathers()
    fire(2, 0)
    transpose(1)
    start_out(1, 1)

    def pair(p, carry):
        i = 2 * p + 2
        drain_gathers()
        fire(i + 1, 1)
        drain_out(0)
        transpose(0)
        start_out(i, 0)

        drain_gathers()
        fire(i + 2, 0)
        drain_out(1)
        transpose(1)
        start_out(i + 1, 1)
        return carry

    lax.fori_loop(0, (TPW - 4) // 2, pair, 0)

    drain_gathers()
    fire(TPW - 1, 1)
    drain_out(0)
    transpose(0)
    start_out(TPW - 2, 0)

    drain_gathers()
    drain_out(1)
    transpose(1)
    start_out(TPW - 1, 1)

    drain_out(0)
    drain_out(1)


def kernel(inputs, embedding):
    idx2d = inputs.T.reshape(N_ROWS, IDX_W)
    tail_p = lax.slice(embedding, (TAIL_C0, 0), (NUM_EMB, FEAT)).reshape(
        TAIL_C // PACK, PACK * FEAT
    )
    packed = _repack_sc(embedding.T, tail_p)
    table_rm = packed.reshape(NUM_EMB, FEAT)
    out = _embed_sc(idx2d, table_rm)
    return jnp.transpose(out, (2, 0, 1))


# speedup vs baseline: 4.4934x; 1.0337x over previous
import functools

import jax
import jax.numpy as jnp
from jax import lax
from jax.experimental import pallas as pl
from jax.experimental.pallas import tpu as pltpu
from jax.experimental.pallas import tpu_sc as plsc

NUM_EMB = 1000000
FEAT = 32
PACK = 4
N_B = 16384
N_J = 26
B_TOTAL = N_B * N_J
IDX_W = 128
N_ROWS = B_TOTAL // IDX_W
NC, NS = 2, 16
NW = NC * NS
L = 16

C_BLK = 512
N_CBLK = NUM_EMB // C_BLK
RB = (N_CBLK + NW - 1) // NW
TAIL_C0 = N_CBLK * C_BLK
TAIL_C = NUM_EMB - TAIL_C0

B_BLK = 512
GPT = B_BLK // IDX_W
N_BLK = N_B // B_BLK
N_TASKS = N_J * N_BLK
TPW = N_TASKS // NW
K = TPW * GPT

_SC_PARAMS = pltpu.CompilerParams(
    use_tc_tiling_on_sc=False, needs_layout_passes=False
)


@functools.partial(
    pl.kernel,
    out_type=jax.ShapeDtypeStruct((NUM_EMB // PACK, PACK * FEAT), jnp.float32),
    mesh=plsc.VectorSubcoreMesh(core_axis_name="c", subcore_axis_name="s"),
    scratch_types=[
        pltpu.VMEM((2, FEAT, C_BLK), jnp.float32),
        pltpu.VMEM((2, C_BLK // PACK, PACK * FEAT), jnp.float32),
        pltpu.SemaphoreType.DMA,
        pltpu.SemaphoreType.DMA,
        pltpu.SemaphoreType.DMA,
    ],
    compiler_params=pltpu.CompilerParams(
        use_tc_tiling_on_sc=True, needs_layout_passes=False
    ),
)
def _repack_sc(tab_t_hbm, tail_hbm, out_hbm, cin_v, cout_v, sem_i, sem_o0, sem_o1):
    wid = lax.axis_index("s") * NC + lax.axis_index("c")
    sems = (sem_o0, sem_o1)

    def c0_of(k):
        m = wid + NW * k
        m = jnp.where(m >= N_CBLK, m - NW, m)
        return m * C_BLK

    def fire_in(k, buf):
        pltpu.async_copy(
            tab_t_hbm.at[:, pl.ds(pl.multiple_of(c0_of(k), 128), C_BLK)],
            cin_v.at[buf],
            sem_i,
        )

    def drain_in():
        pltpu.make_async_copy(
            tab_t_hbm.at[:, pl.ds(0, C_BLK)], cin_v.at[0], sem_i
        ).wait()

    f_lo = lax.iota(jnp.int32, L)
    f_hi = f_lo + L

    def transpose(buf):
        cin2 = cin_v.at[buf]

        @plsc.parallel_loop(0, C_BLK // PACK, unroll=4)
        def _(q):
            base = jnp.full((L,), PACK * q, jnp.int32)
            for k in range(PACK):
                rs = base + k
                x0 = plsc.load_gather(cin2, [f_lo, rs])
                x1 = plsc.load_gather(cin2, [f_hi, rs])
                cout_v[buf, q, pl.ds(k * FEAT, L)] = x0
                cout_v[buf, q, pl.ds(k * FEAT + L, L)] = x1

    def fire_out(k, buf):
        pltpu.async_copy(
            cout_v.at[buf],
            out_hbm.at[pl.ds(pl.multiple_of(c0_of(k) // PACK, 8), C_BLK // PACK)],
            sems[buf],
        )

    def drain_out(buf):
        pltpu.make_async_copy(
            cout_v.at[0], out_hbm.at[pl.ds(0, C_BLK // PACK)], sems[buf]
        ).wait()

    fire_in(0, 0)
    drain_in()
    fire_in(1, 1)
    transpose(0)
    fire_out(0, 0)

    drain_in()
    fire_in(2, 0)
    transpose(1)
    fire_out(1, 1)

    def pair(p, carry):
        k = 2 * p + 2
        drain_in()
        fire_in(k + 1, 1)
        drain_out(0)
        transpose(0)
        fire_out(k, 0)

        drain_in()
        fire_in(k + 2, 0)
        drain_out(1)
        transpose(1)
        fire_out(k + 1, 1)
        return carry

    lax.fori_loop(0, (RB - 4) // 2, pair, 0)

    drain_in()
    fire_in(RB - 1, 1)
    drain_out(0)
    transpose(0)
    fire_out(RB - 2, 0)

    drain_in()
    drain_out(1)
    transpose(1)
    fire_out(RB - 1, 1)

    drain_out(0)
    drain_out(1)

    @pl.when(wid == 0)
    def _tail():
        pltpu.sync_copy(
            tail_hbm,
            out_hbm.at[pl.ds(TAIL_C0 // PACK, TAIL_C // PACK)],
        )


@functools.partial(
    pl.kernel,
    out_type=jax.ShapeDtypeStruct((N_J, FEAT, N_B), jnp.float32),
    mesh=plsc.VectorSubcoreMesh(core_axis_name="c", subcore_axis_name="s"),
    scratch_types=[
        pltpu.VMEM((K, IDX_W), jnp.int32),
        pltpu.VMEM((2, B_BLK, FEAT), jnp.float32),
        pltpu.VMEM((2, FEAT, B_BLK), jnp.float32),
        pltpu.SemaphoreType.DMA,
        pltpu.SemaphoreType.DMA,
        pltpu.SemaphoreType.DMA,
    ],
    compiler_params=_SC_PARAMS,
)
def _embed_sc(
    idx_hbm, table_hbm, out_hbm, idx_v, rows_v, tr_v, sem_g, sem_o0, sem_o1
):
    wid = lax.axis_index("s") * NC + lax.axis_index("c")
    pltpu.sync_copy(idx_hbm.at[pl.ds(wid * K, K)], idx_v)
    t0 = wid * TPW
    sems = (sem_o0, sem_o1)

    def fire(i, buf):
        for r in range(GPT):
            pltpu.async_copy(
                table_hbm.at[idx_v.at[i * GPT + r]],
                rows_v.at[buf, pl.ds(r * IDX_W, IDX_W)],
                sem_g,
            )

    def drain_gathers():
        pltpu.make_async_copy(
            table_hbm.at[pl.ds(0, B_BLK)], rows_v.at[0], sem_g
        ).wait()

    b_iota = lax.iota(jnp.int32, L)

    def transpose(buf):
        rows2 = rows_v.at[buf]

        @plsc.parallel_loop(0, B_BLK // L, unroll=2)
        def _(g):
            bb = b_iota + g * L
            for f in range(FEAT):
                fs = jnp.full((L,), f, jnp.int32)
                x = plsc.load_gather(rows2, [bb, fs])
                tr_v[buf, f, pl.ds(g * L, L)] = x

    def start_out(i, buf):
        t = t0 + i
        j = t // N_BLK
        blk = t % N_BLK
        pltpu.async_copy(
            tr_v.at[buf],
            out_hbm.at[j, :, pl.ds(blk * B_BLK, B_BLK)],
            sems[buf],
        )

    def drain_out(buf):
        pltpu.make_async_copy(
            tr_v.at[0], out_hbm.at[0, :, pl.ds(0, B_BLK)], sems[buf]
        ).wait()

    fire(0, 0)
    drain_gathers()
    fire(1, 1)
    transpose(0)
    start_out(0, 0)

    drain_gathers()
    fire(2, 0)
    transpose(1)
    start_out(1, 1)

    def pair(p, carry):
        i = 2 * p + 2
        drain_gathers()
        fire(i + 1, 1)
        drain_out(0)
        transpose(0)
        start_out(i, 0)

        drain_gathers()
        fire(i + 2, 0)
        drain_out(1)
        transpose(1)
        start_out(i + 1, 1)
        return carry

    lax.fori_loop(0, (TPW - 4) // 2, pair, 0)

    drain_gathers()
    fire(TPW - 1, 1)
    drain_out(0)
    transpose(0)
    start_out(TPW - 2, 0)

    drain_gathers()
    drain_out(1)
    transpose(1)
    start_out(TPW - 1, 1)

    drain_out(0)
    drain_out(1)


def kernel(inputs, embedding):
    idx2d = inputs.T.reshape(N_ROWS, IDX_W)
    tail_p = lax.slice(embedding, (TAIL_C0, 0), (NUM_EMB, FEAT)).reshape(
        TAIL_C // PACK, PACK * FEAT
    )
    packed = _repack_sc(embedding.T, tail_p)
    table_rm = packed.reshape(NUM_EMB, FEAT)
    out = _embed_sc(idx2d, table_rm)
    return jnp.transpose(out, (2, 0, 1))


# speedup vs baseline: 5.6411x; 1.2554x over previous
import functools

import jax
import jax.numpy as jnp
from jax import lax
from jax.experimental import pallas as pl
from jax.experimental.pallas import tpu as pltpu
from jax.experimental.pallas import tpu_sc as plsc

NUM_EMB = 1000000
FEAT = 32
PACK = 4
N_B = 16384
N_J = 26
B_TOTAL = N_B * N_J
IDX_W = 128
N_ROWS = B_TOTAL // IDX_W
NC, NS = 2, 16
NW = NC * NS
L = 16

C_BLK = 512
N_CBLK = NUM_EMB // C_BLK
RB = (N_CBLK + NW - 1) // NW
TAIL_C0 = N_CBLK * C_BLK
TAIL_C = NUM_EMB - TAIL_C0

B_BLK = 512
GPT = B_BLK // IDX_W
N_BLK = N_B // B_BLK
N_TASKS = N_J * N_BLK
TPW = N_TASKS // NW
K = TPW * GPT

_SC_PARAMS = pltpu.CompilerParams(
    use_tc_tiling_on_sc=False, needs_layout_passes=False
)


@functools.partial(
    pl.kernel,
    out_type=jax.ShapeDtypeStruct((NUM_EMB // PACK, PACK * FEAT), jnp.float32),
    mesh=plsc.VectorSubcoreMesh(core_axis_name="c", subcore_axis_name="s"),
    scratch_types=[
        pltpu.VMEM((2, FEAT, C_BLK), jnp.float32),
        pltpu.VMEM((2, C_BLK // PACK, PACK * FEAT), jnp.float32),
        pltpu.SemaphoreType.DMA,
        pltpu.SemaphoreType.DMA,
        pltpu.SemaphoreType.DMA,
    ],
    compiler_params=pltpu.CompilerParams(
        use_tc_tiling_on_sc=True, needs_layout_passes=False
    ),
)
def _repack_sc(tab_t_hbm, tail_hbm, out_hbm, cin_v, cout_v, sem_i, sem_o0, sem_o1):
    wid = lax.axis_index("s") * NC + lax.axis_index("c")
    sems = (sem_o0, sem_o1)

    def c0_of(k):
        m = wid + NW * k
        m = jnp.where(m >= N_CBLK, m - NW, m)
        return m * C_BLK

    def fire_in(k, buf):
        pltpu.async_copy(
            tab_t_hbm.at[:, pl.ds(pl.multiple_of(c0_of(k), 128), C_BLK)],
            cin_v.at[buf],
            sem_i,
        )

    def drain_in():
        pltpu.make_async_copy(
            tab_t_hbm.at[:, pl.ds(0, C_BLK)], cin_v.at[0], sem_i
        ).wait()

    f_lo = lax.iota(jnp.int32, L)
    f_hi = f_lo + L

    def transpose(buf):
        cin2 = cin_v.at[buf]

        @plsc.parallel_loop(0, C_BLK // PACK, unroll=4)
        def _(q):
            base = jnp.full((L,), PACK * q, jnp.int32)
            for k in range(PACK):
                rs = base + k
                x0 = plsc.load_gather(cin2, [f_lo, rs])
                x1 = plsc.load_gather(cin2, [f_hi, rs])
                cout_v[buf, q, pl.ds(k * FEAT, L)] = x0
                cout_v[buf, q, pl.ds(k * FEAT + L, L)] = x1

    def fire_out(k, buf):
        pltpu.async_copy(
            cout_v.at[buf],
            out_hbm.at[pl.ds(pl.multiple_of(c0_of(k) // PACK, 8), C_BLK // PACK)],
            sems[buf],
        )

    def drain_out(buf):
        pltpu.make_async_copy(
            cout_v.at[0], out_hbm.at[pl.ds(0, C_BLK // PACK)], sems[buf]
        ).wait()

    fire_in(0, 0)
    drain_in()
    fire_in(1, 1)
    transpose(0)
    fire_out(0, 0)

    drain_in()
    fire_in(2, 0)
    transpose(1)
    fire_out(1, 1)

    def pair(p, carry):
        k = 2 * p + 2
        drain_in()
        fire_in(k + 1, 1)
        drain_out(0)
        transpose(0)
        fire_out(k, 0)

        drain_in()
        fire_in(k + 2, 0)
        drain_out(1)
        transpose(1)
        fire_out(k + 1, 1)
        return carry

    lax.fori_loop(0, (RB - 4) // 2, pair, 0)

    drain_in()
    fire_in(RB - 1, 1)
    drain_out(0)
    transpose(0)
    fire_out(RB - 2, 0)

    drain_in()
    drain_out(1)
    transpose(1)
    fire_out(RB - 1, 1)

    drain_out(0)
    drain_out(1)

    @pl.when(wid == 0)
    def _tail():
        pltpu.sync_copy(
            tail_hbm,
            out_hbm.at[pl.ds(TAIL_C0 // PACK, TAIL_C // PACK)],
        )


@functools.partial(
    pl.kernel,
    out_type=jax.ShapeDtypeStruct((N_J, FEAT, N_B), jnp.float32),
    mesh=plsc.VectorSubcoreMesh(core_axis_name="c", subcore_axis_name="s"),
    scratch_types=[
        pltpu.VMEM((K, IDX_W), jnp.int32),
        pltpu.VMEM((2, B_BLK, FEAT), jnp.float32),
        pltpu.VMEM((2, FEAT, B_BLK), jnp.float32),
        pltpu.SemaphoreType.DMA,
        pltpu.SemaphoreType.DMA,
        pltpu.SemaphoreType.DMA,
    ],
    compiler_params=_SC_PARAMS,
)
def _embed_sc(
    idx_hbm, table_hbm, out_hbm, idx_v, rows_v, tr_v, sem_g, sem_o0, sem_o1
):
    wid = lax.axis_index("s") * NC + lax.axis_index("c")
    pltpu.sync_copy(idx_hbm.at[pl.ds(wid * K, K)], idx_v)
    t0 = wid * TPW
    sems = (sem_o0, sem_o1)

    def fire(i, buf):
        for r in range(GPT):
            pltpu.async_copy(
                table_hbm.at[idx_v.at[i * GPT + r]],
                rows_v.at[buf, pl.ds(r * IDX_W, IDX_W)],
                sem_g,
            )

    def drain_gathers():
        pltpu.make_async_copy(
            table_hbm.at[pl.ds(0, B_BLK)], rows_v.at[0], sem_g
        ).wait()

    b_iota = lax.iota(jnp.int32, L)

    diags = [((b_iota + s) % L) + h * L for h in range(2) for s in range(L)]

    def transpose(buf):
        rows2 = rows_v.at[buf]
        tr2 = tr_v.at[buf]

        @plsc.parallel_loop(0, B_BLK // L, unroll=2)
        def _(g):
            bb = b_iota + g * L
            for ff in diags:
                x = plsc.load_gather(rows2, [bb, ff])
                plsc.store_scatter(tr2, [ff, bb], x)

    def start_out(i, buf):
        t = t0 + i
        j = t // N_BLK
        blk = t % N_BLK
        pltpu.async_copy(
            tr_v.at[buf],
            out_hbm.at[j, :, pl.ds(blk * B_BLK, B_BLK)],
            sems[buf],
        )

    def drain_out(buf):
        pltpu.make_async_copy(
            tr_v.at[0], out_hbm.at[0, :, pl.ds(0, B_BLK)], sems[buf]
        ).wait()

    fire(0, 0)
    drain_gathers()
    fire(1, 1)
    transpose(0)
    start_out(0, 0)

    drain_gathers()
    fire(2, 0)
    transpose(1)
    start_out(1, 1)

    def pair(p, carry):
        i = 2 * p + 2
        drain_gathers()
        fire(i + 1, 1)
        drain_out(0)
        transpose(0)
        start_out(i, 0)

        drain_gathers()
        fire(i + 2, 0)
        drain_out(1)
        transpose(1)
        start_out(i + 1, 1)
        return carry

    lax.fori_loop(0, (TPW - 4) // 2, pair, 0)

    drain_gathers()
    fire(TPW - 1, 1)
    drain_out(0)
    transpose(0)
    start_out(TPW - 2, 0)

    drain_gathers()
    drain_out(1)
    transpose(1)
    start_out(TPW - 1, 1)

    drain_out(0)
    drain_out(1)


def kernel(inputs, embedding):
    idx2d = inputs.T.reshape(N_ROWS, IDX_W)
    tail_p = lax.slice(embedding, (TAIL_C0, 0), (NUM_EMB, FEAT)).reshape(
        TAIL_C // PACK, PACK * FEAT
    )
    packed = _repack_sc(embedding.T, tail_p)
    table_rm = packed.reshape(NUM_EMB, FEAT)
    out = _embed_sc(idx2d, table_rm)
    return jnp.transpose(out, (2, 0, 1))
